# Initial kernel scaffold; baseline (speedup 1.0000x reference)
import jax
import jax.numpy as jnp
from jax.experimental import pallas as pl


def kernel(tokens, weight):
    raise NotImplementedError("write your pallas kernel here")


# speedup vs baseline: 4.0074x; 4.0074x over previous
import functools
import jax
import jax.numpy as jnp
from jax import lax
from jax.experimental import pallas as pl
from jax.experimental.pallas import tpu as pltpu
from jax.experimental.pallas import tpu_sc as plsc

D_MODEL_K = 32
B_TOTAL = 16384 * 200

_info = plsc.get_sparse_core_info()
_NC, _NS = _info.num_cores, _info.num_subcores
_NW = _NC * _NS

_PER_W = B_TOTAL // _NW
_CHUNK = 2048
_STEPS = _PER_W // _CHUNK


def _emb_body(tokens_hbm, table_hbm, out_hbm, idx_v, rows_v, sem):
    wid = lax.axis_index("s") * _NC + lax.axis_index("c")
    w_base = wid * _PER_W

    def step(g, carry):
        base = w_base + g * _CHUNK
        pltpu.sync_copy(tokens_hbm.at[pl.ds(base, _CHUNK)], idx_v)
        pltpu.async_copy(table_hbm.at[idx_v], rows_v, sem).wait()
        pltpu.sync_copy(rows_v, out_hbm.at[pl.ds(base, _CHUNK)])
        return carry

    lax.fori_loop(0, _STEPS, step, 0)


@jax.jit
def _emb_call(tokens_flat, weight):
    mesh = plsc.VectorSubcoreMesh(core_axis_name="c", subcore_axis_name="s")
    f = pl.kernel(
        _emb_body,
        out_type=jax.ShapeDtypeStruct((B_TOTAL, D_MODEL_K), jnp.float32),
        mesh=mesh,
        scratch_types=[
            pltpu.VMEM((_CHUNK,), jnp.int32),
            pltpu.VMEM((_CHUNK, D_MODEL_K), jnp.float32),
            pltpu.SemaphoreType.DMA,
        ],
        compiler_params=pltpu.CompilerParams(use_tc_tiling_on_sc=False),
    )
    return f(tokens_flat, weight)


def kernel(tokens, weight):
    tokens_flat = tokens.reshape(B_TOTAL).astype(jnp.int32)
    out = _emb_call(tokens_flat, weight)
    return out.reshape(*tokens.shape, D_MODEL_K)
